# baseline (speedup 1.0000x reference)
import functools

import jax
import jax.numpy as jnp
from jax import lax
from jax.experimental import pallas as pl
from jax.experimental.pallas import tpu as pltpu
from jax.experimental.pallas import tpu_sc as plsc

B = 16384
D = 64
NC = 2
NS = 16
NW = NC * NS
BPW = B // NW
CHUNK = 128
NCHUNK = BPW // CHUNK


def _sc_gather(user2d, item2d, user_table, item_table):
    mesh = plsc.VectorSubcoreMesh(core_axis_name="c", subcore_axis_name="s")

    @functools.partial(
        pl.kernel,
        mesh=mesh,
        compiler_params=pltpu.CompilerParams(use_tc_tiling_on_sc=False),
        out_type=(
            jax.ShapeDtypeStruct((B, D), jnp.float32),
            jax.ShapeDtypeStruct((B, D), jnp.float32),
        ),
        scratch_types=[
            pltpu.VMEM((NCHUNK, CHUNK), jnp.int32),
            pltpu.VMEM((NCHUNK, CHUNK), jnp.int32),
            pltpu.VMEM((BPW, D), jnp.float32),
            pltpu.VMEM((BPW, D), jnp.float32),
            pltpu.SemaphoreType.DMA,
            pltpu.SemaphoreType.DMA,
        ],
    )
    def k(u_hbm, i_hbm, ut_hbm, it_hbm, qo_hbm, co_hbm,
          uix_v, iix_v, q_v, c_v, sem_q, sem_c):
        wid = lax.axis_index("s") * NC + lax.axis_index("c")
        base = wid * BPW
        pltpu.sync_copy(u_hbm.at[pl.ds(wid * NCHUNK, NCHUNK)], uix_v)
        pltpu.sync_copy(i_hbm.at[pl.ds(wid * NCHUNK, NCHUNK)], iix_v)
        copies = []
        for g in range(NCHUNK):
            copies.append(pltpu.async_copy(
                ut_hbm.at[uix_v.at[g]],
                q_v.at[pl.ds(g * CHUNK, CHUNK)], sem_q))
            copies.append(pltpu.async_copy(
                it_hbm.at[iix_v.at[g]],
                c_v.at[pl.ds(g * CHUNK, CHUNK)], sem_c))
        for cp in copies:
            cp.wait()
        pltpu.sync_copy(q_v, qo_hbm.at[pl.ds(base, BPW)])
        pltpu.sync_copy(c_v, co_hbm.at[pl.ds(base, BPW)])

    return k(user2d, item2d, user_table, item_table)


def _tc_cosine(q, c):
    def body(q_ref, c_ref, o_ref):
        qv = q_ref[...]
        cv = c_ref[...]
        eps = jnp.float32(1e-8)
        dot = jnp.sum(qv * cv, axis=-1)
        qn = jnp.maximum(jnp.sqrt(jnp.sum(qv * qv, axis=-1)), eps)
        cn = jnp.maximum(jnp.sqrt(jnp.sum(cv * cv, axis=-1)), eps)
        o_ref[...] = dot / (qn * cn)

    return pl.pallas_call(
        body,
        out_shape=jax.ShapeDtypeStruct((B,), jnp.float32),
    )(q, c)


def kernel(user, item, user_table, item_table):
    user2d = user.reshape(NW * NCHUNK, CHUNK)
    item2d = item.reshape(NW * NCHUNK, CHUNK)
    q, c = _sc_gather(user2d, item2d, user_table, item_table)
    return _tc_cosine(q, c)
